# Initial kernel scaffold; baseline (speedup 1.0000x reference)
import jax
import jax.numpy as jnp
from jax.experimental import pallas as pl


def kernel(h, positions, past_positions, W, b):
    raise NotImplementedError("write your pallas kernel here")


# speedup vs baseline: 73.5600x; 73.5600x over previous
import functools

import jax
import jax.numpy as jnp
from jax import lax
from jax.experimental import pallas as pl
from jax.experimental.pallas import tpu as pltpu
from jax.experimental.pallas import tpu_sc as plsc

_P = 4096
_NG = 32
_NBINS = _NG * _NG
_ODIM = 128
_WORKERS = 32
_PEDS_PER_TILE = _P // _WORKERS
_L = 16
_GROUPS = _PEDS_PER_TILE // _L


def _occ_body(xs_hbm, ys_hbm, occ_hbm, xs, ys, hist):
    wid = lax.axis_index("c") * 16 + lax.axis_index("s")
    pltpu.sync_copy(xs_hbm, xs)
    pltpu.sync_copy(ys_hbm, ys)
    lanes = lax.iota(jnp.int32, _L)
    hist_base = lanes * _NBINS
    ones = jnp.ones((_L,), jnp.float32)
    zeros16 = jnp.zeros((_L,), jnp.float32)

    for g in range(_GROUPS):
        pbase = wid * _PEDS_PER_TILE + g * _L
        vx = xs[pl.ds(pbase, _L)]
        vy = ys[pl.ds(pbase, _L)]

        def _zero(k, _):
            hist[pl.ds(k * _L, _L)] = zeros16
            return 0

        lax.fori_loop(0, (_L * _NBINS) // _L, _zero, 0)

        def _pair_chunk(jc, _):
            vxj = xs[pl.ds(jc * _L, _L)]
            vyj = ys[pl.ds(jc * _L, _L)]
            for k in range(_L):
                fx = (vxj[k] - vx) * 16.0 + 16.0
                fy = (vyj[k] - vy) * 16.0 + 16.0
                ox = jnp.minimum(fx.astype(jnp.int32), _NG - 1)
                oy = jnp.minimum(fy.astype(jnp.int32), _NG - 1)
                idx = hist_base + ox * _NG + oy
                plsc.addupdate_scatter(hist, [idx], ones)
            return 0

        lax.fori_loop(0, _P // _L, _pair_chunk, 0)

        plsc.addupdate_scatter(hist, [hist_base + (_NG // 2) * _NG + _NG // 2], -ones)

        pltpu.sync_copy(hist, occ_hbm.at[pl.ds(pbase * _NBINS, _L * _NBINS)])


def _occupancy_sc(xs, ys):
    mesh = plsc.VectorSubcoreMesh(core_axis_name="c", subcore_axis_name="s")
    fn = pl.kernel(
        _occ_body,
        mesh=mesh,
        out_type=jax.ShapeDtypeStruct((_P * _NBINS,), jnp.float32),
        scratch_types=[
            pltpu.VMEM((_P,), jnp.float32),
            pltpu.VMEM((_P,), jnp.float32),
            pltpu.VMEM((_L * _NBINS,), jnp.float32),
        ],
        compiler_params=pltpu.CompilerParams(needs_layout_passes=False),
    )
    return fn(xs, ys)


def _embed_body(occ_ref, w_ref, b_ref, out_ref):
    acc = jnp.dot(occ_ref[...], w_ref[...], preferred_element_type=jnp.float32)
    out_ref[...] = jnp.maximum(acc + b_ref[...], 0.0)


def _embed_tc(occ, W, b):
    bm = 512
    return pl.pallas_call(
        _embed_body,
        grid=(_P // bm,),
        in_specs=[
            pl.BlockSpec((bm, _NBINS), lambda i: (i, 0)),
            pl.BlockSpec((_NBINS, _ODIM), lambda i: (0, 0)),
            pl.BlockSpec((1, _ODIM), lambda i: (0, 0)),
        ],
        out_specs=pl.BlockSpec((bm, _ODIM), lambda i: (i, 0)),
        out_shape=jax.ShapeDtypeStruct((_P, _ODIM), jnp.float32),
    )(occ, W, b.reshape(1, _ODIM))


def kernel(h, positions, past_positions, W, b):
    xs = positions[:, 0]
    ys = positions[:, 1]
    occ = _occupancy_sc(xs, ys).reshape(_P, _NBINS)
    return _embed_tc(occ, W, b)


# speedup vs baseline: 79.1171x; 1.0755x over previous
import functools

import jax
import jax.numpy as jnp
from jax import lax
from jax.experimental import pallas as pl
from jax.experimental.pallas import tpu as pltpu
from jax.experimental.pallas import tpu_sc as plsc

_P = 4096
_NG = 32
_NBINS = _NG * _NG
_ODIM = 128
_WORKERS = 32
_PEDS_PER_TILE = _P // _WORKERS
_L = 16
_GROUPS = _PEDS_PER_TILE // _L


_GIN = 4
_PASSES = _GROUPS // _GIN


def _occ_body(xs_hbm, ys_hbm, occ_hbm, xs, ys, sx16, sy16, hist):
    wid = lax.axis_index("c") * 16 + lax.axis_index("s")
    pltpu.sync_copy(xs_hbm, xs)
    pltpu.sync_copy(ys_hbm, ys)
    lanes = lax.iota(jnp.int32, _L)
    ones = jnp.ones((_L,), jnp.float32)
    zeros16 = jnp.zeros((_L,), jnp.float32)

    def _scale(k, _):
        sx16[pl.ds(k * _L, _L)] = xs[pl.ds(k * _L, _L)] * 16.0 + 16.0
        sy16[pl.ds(k * _L, _L)] = ys[pl.ds(k * _L, _L)] * 16.0 + 16.0
        return 0

    lax.fori_loop(0, _P // _L, _scale, 0)

    for p in range(_PASSES):
        pbase = wid * _PEDS_PER_TILE + p * (_GIN * _L)
        vx16 = []
        vy16 = []
        bases = []
        for g in range(_GIN):
            vx16.append(xs[pl.ds(pbase + g * _L, _L)] * 16.0)
            vy16.append(ys[pl.ds(pbase + g * _L, _L)] * 16.0)
            bases.append(lanes * _NBINS + g * (_L * _NBINS))

        def _zero(k, _):
            hist[pl.ds(k * _L, _L)] = zeros16
            return 0

        lax.fori_loop(0, (_GIN * _L * _NBINS) // _L, _zero, 0)

        def _pair_chunk(jc, _):
            vxj = sx16[pl.ds(jc * _L, _L)]
            vyj = sy16[pl.ds(jc * _L, _L)]
            for k in range(_L):
                bx = jnp.broadcast_to(vxj[k], (_L,))
                by = jnp.broadcast_to(vyj[k], (_L,))
                for g in range(_GIN):
                    fx = bx - vx16[g]
                    fy = by - vy16[g]
                    ox = fx.astype(jnp.int32)
                    oy = fy.astype(jnp.int32)
                    idx = jnp.minimum(ox * _NG + oy, _NBINS - 1) + bases[g]
                    plsc.addupdate_scatter(hist, [idx], ones)
            return 0

        lax.fori_loop(0, _P // _L, _pair_chunk, 0)

        for g in range(_GIN):
            plsc.addupdate_scatter(
                hist, [bases[g] + (_NG // 2) * _NG + _NG // 2], -ones
            )

        pltpu.sync_copy(
            hist, occ_hbm.at[pl.ds(pbase * _NBINS, _GIN * _L * _NBINS)]
        )


def _occupancy_sc(xs, ys):
    mesh = plsc.VectorSubcoreMesh(core_axis_name="c", subcore_axis_name="s")
    fn = pl.kernel(
        _occ_body,
        mesh=mesh,
        out_type=jax.ShapeDtypeStruct((_P * _NBINS,), jnp.float32),
        scratch_types=[
            pltpu.VMEM((_P,), jnp.float32),
            pltpu.VMEM((_P,), jnp.float32),
            pltpu.VMEM((_P,), jnp.float32),
            pltpu.VMEM((_P,), jnp.float32),
            pltpu.VMEM((_GIN * _L * _NBINS,), jnp.float32),
        ],
        compiler_params=pltpu.CompilerParams(needs_layout_passes=False),
    )
    return fn(xs, ys)


def _embed_body(occ_ref, w_ref, b_ref, out_ref):
    acc = jnp.dot(occ_ref[...], w_ref[...], preferred_element_type=jnp.float32)
    out_ref[...] = jnp.maximum(acc + b_ref[...], 0.0)


def _embed_tc(occ, W, b):
    bm = 512
    return pl.pallas_call(
        _embed_body,
        grid=(_P // bm,),
        in_specs=[
            pl.BlockSpec((bm, _NBINS), lambda i: (i, 0)),
            pl.BlockSpec((_NBINS, _ODIM), lambda i: (0, 0)),
            pl.BlockSpec((1, _ODIM), lambda i: (0, 0)),
        ],
        out_specs=pl.BlockSpec((bm, _ODIM), lambda i: (i, 0)),
        out_shape=jax.ShapeDtypeStruct((_P, _ODIM), jnp.float32),
    )(occ, W, b.reshape(1, _ODIM))


def kernel(h, positions, past_positions, W, b):
    xs = positions[:, 0]
    ys = positions[:, 1]
    occ = _occupancy_sc(xs, ys).reshape(_P, _NBINS)
    return _embed_tc(occ, W, b)
